# Initial kernel scaffold; baseline (speedup 1.0000x reference)
import jax
import jax.numpy as jnp
from jax.experimental import pallas as pl


def kernel(x, edge_index, batch, hyperparameters, node_emb, msg_W0, msg_b0, msgr_W0, msgr_b0, gru_wih0, gru_whh0, gru_bih0, gru_bhh0, msg_W1, msg_b1, msgr_W1, msgr_b1, gru_wih1, gru_whh1, gru_bih1, gru_bhh1, msg_W2, msg_b2, msgr_W2, msgr_b2, gru_wih2, gru_whh2, gru_bih2, gru_bhh2, fc1_W, fc1_b, fm_W, fm_b, gm_W, gm_b, acc_W0, acc_b0, acc_W1, acc_b1, acc_W2, acc_b2):
    raise NotImplementedError("write your pallas kernel here")


# speedup vs baseline: 8.7641x; 8.7641x over previous
import functools

import jax
import jax.numpy as jnp
from jax import lax
from jax.experimental import pallas as pl
from jax.experimental.pallas import tpu as pltpu
from jax.experimental.pallas import tpu_sc as plsc

N = 7680
E = 122880
G = 256
ND = 128

_F32 = jnp.float32
_HI = lax.Precision.HIGHEST

_CHUNK = 128
_EPW = E // 16
_NCH = _EPW // _CHUNK
_RPS = N // 16

def _dot(a, b):
    return jnp.dot(a, b, precision=_HI, preferred_element_type=_F32)


def _r(x):
    return x.astype(jnp.bfloat16).astype(_F32)


@functools.cache
def _mesh():
    return plsc.VectorSubcoreMesh(core_axis_name="c", subcore_axis_name="s",
                                  num_cores=2, num_subcores=16)


@functools.cache
def _spmm_kernel():
  @functools.partial(
      pl.kernel,
      out_type=(jax.ShapeDtypeStruct((N, ND), _F32),
                jax.ShapeDtypeStruct((N, ND), _F32)),
      mesh=_mesh(),
      scratch_types=[
          pltpu.VMEM((_CHUNK,), jnp.int32),
          pltpu.VMEM((_CHUNK,), jnp.int32),
          pltpu.VMEM((_CHUNK, ND), _F32),
          pltpu.VMEM_SHARED((N, ND), _F32),
          pltpu.SemaphoreType.DMA,
      ],
  )
  def _spmm(h_hbm, src_hbm, dst_hbm, zeros_hbm, p_hbm, q_hbm,
            gidx, sidx, buf, acc, sem):
    cid = lax.axis_index("c")
    sid = lax.axis_index("s")
    r0 = sid * _RPS
    pltpu.sync_copy(zeros_hbm.at[pl.ds(r0, _RPS)], acc.at[pl.ds(r0, _RPS)])
    plsc.subcore_barrier()
    base = sid * _EPW

    def run(g_hbm, s_hbm, out_hbm):
        def body(k, carry):
            off = base + k * _CHUNK
            pltpu.sync_copy(g_hbm.at[pl.ds(off, _CHUNK)], gidx)
            pltpu.sync_copy(s_hbm.at[pl.ds(off, _CHUNK)], sidx)
            pltpu.async_copy(h_hbm.at[gidx], buf, sem).wait()
            pltpu.sync_copy(buf, acc.at[sidx], add=True)
            return carry
        lax.fori_loop(0, _NCH, body, 0)
        plsc.subcore_barrier()
        pltpu.sync_copy(acc.at[pl.ds(r0, _RPS)], out_hbm.at[pl.ds(r0, _RPS)])

    @pl.when(cid == 0)
    def _():
        run(src_hbm, dst_hbm, p_hbm)

    @pl.when(cid == 1)
    def _():
        run(dst_hbm, src_hbm, q_hbm)

  return _spmm


def _spmm(h, src, dst, zeros_nd):
    return _spmm_kernel()(h, src, dst, zeros_nd)


def _embed_call(x3, emb16):
    def body(x_ref, emb_ref, o_ref, or_ref):
        xv = x_ref[0, 0, :]
        oh = (xv[:, None] == lax.broadcasted_iota(jnp.int32, (256, 16), 1))
        h = _dot(oh.astype(_F32), emb_ref[...])
        o_ref[...] = h
        or_ref[...] = _r(h)

    return pl.pallas_call(
        body,
        grid=(N // 256,),
        in_specs=[pl.BlockSpec((1, 1, 256), lambda i: (i, 0, 0)),
                  pl.BlockSpec((16, ND), lambda i: (0, 0))],
        out_specs=[pl.BlockSpec((256, ND), lambda i: (i, 0)),
                   pl.BlockSpec((256, ND), lambda i: (i, 0))],
        out_shape=[jax.ShapeDtypeStruct((N, ND), _F32),
                   jax.ShapeDtypeStruct((N, ND), _F32)],
    )(x3, emb16)


def _update_call(P, Q, h, hr, ind16, outd16, waT, wraT, wbT, wrbT, msgb,
                 msgrb, wihT, whhT, bih, bhh):
    R = 256

    def body(p_ref, q_ref, h_ref, hr_ref, ind_ref, outd_ref, waT_ref,
             wraT_ref, wbT_ref, wrbT_ref, msgb_ref, msgrb_ref, wihT_ref,
             whhT_ref, bih_ref, bhh_ref, o_ref, or_ref):
        hb = h_ref[...]
        hrb = hr_ref[...]
        ind = ind_ref[:, :1]
        outd = outd_ref[:, :1]
        aggr = (_dot(p_ref[...], waT_ref[...]) + _dot(q_ref[...], wraT_ref[...])
                + ind * _dot(hrb, wbT_ref[...]) + outd * _dot(hrb, wrbT_ref[...])
                + ind * msgb_ref[...] + outd * msgrb_ref[...])
        gi = _dot(_r(aggr), wihT_ref[...]) + bih_ref[...]
        gh = _dot(hrb, whhT_ref[...]) + bhh_ref[...]
        r = jax.nn.sigmoid(gi[:, :ND] + gh[:, :ND])
        z = jax.nn.sigmoid(gi[:, ND:2 * ND] + gh[:, ND:2 * ND])
        n = jnp.tanh(gi[:, 2 * ND:] + r * gh[:, 2 * ND:])
        h_new = (1.0 - z) * n + z * hb
        o_ref[...] = h_new
        or_ref[...] = _r(h_new)

    row = lambda shape: pl.BlockSpec(shape, lambda i: (i, 0))
    cst = lambda shape: pl.BlockSpec(shape, lambda i: (0, 0))
    return pl.pallas_call(
        body,
        grid=(N // R,),
        in_specs=[row((R, ND)), row((R, ND)), row((R, ND)), row((R, ND)),
                  row((R, 16)), row((R, 16)),
                  cst((ND, 2 * ND)), cst((ND, 2 * ND)),
                  cst((ND, 2 * ND)), cst((ND, 2 * ND)),
                  cst((1, 2 * ND)), cst((1, 2 * ND)),
                  cst((2 * ND, 3 * ND)), cst((ND, 3 * ND)),
                  cst((1, 3 * ND)), cst((1, 3 * ND))],
        out_specs=[row((R, ND)), row((R, ND))],
        out_shape=[jax.ShapeDtypeStruct((N, ND), _F32),
                   jax.ShapeDtypeStruct((N, ND), _F32)],
    )(P, Q, h, hr, ind16, outd16, waT, wraT, wbT, wrbT, msgb, msgrb,
      wihT, whhT, bih, bhh)


def _gated_call(h, hps8, S, batch3, fc1T8, fc1b8, fmWhT, fmWpT, fmb, gmWhT,
                gmWpT, gmb8):
    R = 240

    def body(h_ref, hp_ref, s_ref, b_ref, fc1T_ref, fc1b_ref, fmWhT_ref,
             fmWpT_ref, fmb_ref, gmWhT_ref, gmWpT_ref, gmb_ref, o_ref):
        hp = jax.nn.relu(_dot(hp_ref[...], fc1T_ref[...]) + fc1b_ref[...])
        rep = _r(_dot(s_ref[...], hp))
        hb = h_ref[...]
        hv = _dot(hb, fmWhT_ref[...]) + _dot(rep, fmWpT_ref[...]) + fmb_ref[...]
        gl = _dot(hb, gmWhT_ref[...]) + _dot(rep, gmWpT_ref[...]) + gmb_ref[...]
        gated = hv * jax.nn.sigmoid(gl[:, :1])
        bv = b_ref[0, 0, :]
        oh = (bv[:, None] == lax.broadcasted_iota(jnp.int32, (R, G), 1))
        contrib = lax.dot_general(oh.astype(_F32), gated,
                                  (((0,), (0,)), ((), ())),
                                  precision=_HI, preferred_element_type=_F32)
        i = pl.program_id(0)

        @pl.when(i == 0)
        def _():
            o_ref[...] = contrib

        @pl.when(i > 0)
        def _():
            o_ref[...] += contrib

    return pl.pallas_call(
        body,
        grid=(N // R,),
        in_specs=[pl.BlockSpec((R, ND), lambda i: (i, 0)),
                  pl.BlockSpec((8, 8), lambda i: (i, 0)),
                  pl.BlockSpec((R, 8), lambda i: (0, 0)),
                  pl.BlockSpec((1, 1, R), lambda i: (i, 0, 0)),
                  pl.BlockSpec((8, 8), lambda i: (0, 0)),
                  pl.BlockSpec((1, 8), lambda i: (0, 0)),
                  pl.BlockSpec((ND, 256), lambda i: (0, 0)),
                  pl.BlockSpec((8, 256), lambda i: (0, 0)),
                  pl.BlockSpec((1, 256), lambda i: (0, 0)),
                  pl.BlockSpec((ND, 8), lambda i: (0, 0)),
                  pl.BlockSpec((8, 8), lambda i: (0, 0)),
                  pl.BlockSpec((1, 8), lambda i: (0, 0))],
        out_specs=pl.BlockSpec((G, 256), lambda i: (0, 0)),
        out_shape=jax.ShapeDtypeStruct((G, 256), _F32),
    )(h, hps8, S, batch3, fc1T8, fc1b8, fmWhT, fmWpT, fmb, gmWhT, gmWpT, gmb8)


def _head_call(hG, w0T, b0, w1T, b1, w2T8, b28):
    def body(hg_ref, w0_ref, b0_ref, w1_ref, b1_ref, w2_ref, b2_ref, o_ref):
        o = jax.nn.relu(_dot(_r(hg_ref[...]), w0_ref[...]) + b0_ref[...])
        o = jax.nn.relu(_dot(_r(o), w1_ref[...]) + b1_ref[...])
        o_ref[...] = _dot(_r(o), w2_ref[...]) + b2_ref[...]

    return pl.pallas_call(
        body,
        out_shape=jax.ShapeDtypeStruct((G, 8), _F32),
    )(hG, w0T, b0, w1T, b1, w2T8, b28)


def kernel(x, edge_index, batch, hyperparameters, node_emb,
           msg_W0, msg_b0, msgr_W0, msgr_b0, gru_wih0, gru_whh0, gru_bih0, gru_bhh0,
           msg_W1, msg_b1, msgr_W1, msgr_b1, gru_wih1, gru_whh1, gru_bih1, gru_bhh1,
           msg_W2, msg_b2, msgr_W2, msgr_b2, gru_wih2, gru_whh2, gru_bih2, gru_bhh2,
           fc1_W, fc1_b, fm_W, fm_b, gm_W, gm_b,
           acc_W0, acc_b0, acc_W1, acc_b1, acc_W2, acc_b2):
    msg_W = [msg_W0, msg_W1, msg_W2]
    msg_b = [msg_b0, msg_b1, msg_b2]
    msgr_W = [msgr_W0, msgr_W1, msgr_W2]
    msgr_b = [msgr_b0, msgr_b1, msgr_b2]
    gru_wih = [gru_wih0, gru_wih1, gru_wih2]
    gru_whh = [gru_whh0, gru_whh1, gru_whh2]
    gru_bih = [gru_bih0, gru_bih1, gru_bih2]
    gru_bhh = [gru_bhh0, gru_bhh1, gru_bhh2]

    src = edge_index[0].astype(jnp.int32)
    dst = edge_index[1].astype(jnp.int32)

    x3 = x.astype(jnp.int32).reshape(N // 256, 1, 256)
    emb16 = jnp.zeros((16, ND), _F32).at[:11].set(node_emb)
    h, hr = _embed_call(x3, emb16)

    zeros_nd = jnp.zeros((N, ND), _F32)
    ones_nd = jnp.ones((N, ND), _F32)
    ind_full, outd_full = _spmm(ones_nd, src, dst, zeros_nd)
    ind16 = ind_full[:, :16]
    outd16 = outd_full[:, :16]

    for l in range(3):
        P, Q = _spmm(hr, src, dst, zeros_nd)
        h, hr = _update_call(
            P, Q, h, hr, ind16, outd16,
            _r(msg_W[l][:, :ND].T), _r(msgr_W[l][:, :ND].T),
            _r(msg_W[l][:, ND:].T), _r(msgr_W[l][:, ND:].T),
            msg_b[l].reshape(1, -1), msgr_b[l].reshape(1, -1),
            _r(gru_wih[l].T), _r(gru_whh[l].T),
            gru_bih[l].reshape(1, -1), gru_bhh[l].reshape(1, -1))

    hps8 = _r(jnp.zeros((G, 8), _F32).at[:, :5].set(hyperparameters.reshape(G, 5)))
    S = jnp.zeros((240, 8), _F32).at[jnp.arange(240), jnp.arange(240) // 30].set(1.0)
    fc1T8 = _r(jnp.zeros((8, 8), _F32).at[:5, :5].set(fc1_W.T))
    fc1b8 = jnp.zeros((1, 8), _F32).at[0, :5].set(fc1_b)
    fmWhT = _r(fm_W[:, :ND].T)
    fmWpT = _r(jnp.zeros((8, 256), _F32).at[:5].set(fm_W[:, ND:].T))
    fmb = fm_b.reshape(1, -1)
    gmWhT = _r(jnp.zeros((ND, 8), _F32).at[:, :1].set(gm_W[:, :ND].T))
    gmWpT = _r(jnp.zeros((8, 8), _F32).at[:5, :1].set(gm_W[:, ND:].T))
    gmb8 = jnp.zeros((1, 8), _F32).at[0, 0].set(gm_b[0])
    batch3 = batch.astype(jnp.int32).reshape(N // 240, 1, 240)
    hG = _gated_call(hr, hps8, S, batch3, fc1T8, fc1b8, fmWhT, fmWpT, fmb,
                     gmWhT, gmWpT, gmb8)

    w0T, b0 = _r(acc_W0.T), acc_b0.reshape(1, -1)
    w1T, b1 = _r(acc_W1.T), acc_b1.reshape(1, -1)
    w2T8 = _r(jnp.zeros((64, 8), _F32).at[:, :1].set(acc_W2.T))
    b28 = jnp.zeros((1, 8), _F32).at[0, 0].set(acc_b2[0])
    out8 = _head_call(hG, w0T, b0, w1T, b1, w2T8, b28)
    return out8[:, 0]


# speedup vs baseline: 13.0859x; 1.4931x over previous
import functools

import jax
import jax.numpy as jnp
from jax import lax
from jax.experimental import pallas as pl
from jax.experimental.pallas import tpu as pltpu
from jax.experimental.pallas import tpu_sc as plsc

N = 7680
E = 122880
G = 256
ND = 128

_F32 = jnp.float32
_HI = lax.Precision.HIGHEST

_CHUNK = 128
_EPW = E // 16
_NCH = _EPW // _CHUNK
_RPS = N // 16

def _dot(a, b):
    return jnp.dot(a, b, precision=_HI, preferred_element_type=_F32)


def _r(x):
    return x.astype(jnp.bfloat16).astype(_F32)


@functools.cache
def _mesh():
    return plsc.VectorSubcoreMesh(core_axis_name="c", subcore_axis_name="s",
                                  num_cores=2, num_subcores=16)


@functools.cache
def _spmm_kernel():
  @functools.partial(
      pl.kernel,
      out_type=(jax.ShapeDtypeStruct((N, ND), _F32),
                jax.ShapeDtypeStruct((N, ND), _F32)),
      mesh=_mesh(),
      scratch_types=[
          pltpu.VMEM((_NCH, _CHUNK), jnp.int32),
          pltpu.VMEM((_NCH, _CHUNK), jnp.int32),
          pltpu.VMEM((_CHUNK, ND), _F32),
          pltpu.VMEM((_CHUNK, ND), _F32),
          pltpu.VMEM_SHARED((N, ND), _F32),
          pltpu.SemaphoreType.DMA,
          pltpu.SemaphoreType.DMA,
      ],
  )
  def _spmm(h_hbm, src2_hbm, dst2_hbm, zeros_hbm, p_hbm, q_hbm,
            gidx, sidx, buf0, buf1, acc, sem0, sem1):
    cid = lax.axis_index("c")
    sid = lax.axis_index("s")
    r0 = sid * _RPS
    pltpu.sync_copy(zeros_hbm.at[pl.ds(r0, _RPS)], acc.at[pl.ds(r0, _RPS)])
    plsc.subcore_barrier()
    bufs = (buf0, buf1)
    sems = (sem0, sem1)

    def run(g_hbm, s_hbm, out_hbm):
        pltpu.sync_copy(g_hbm.at[sid], gidx)
        pltpu.sync_copy(s_hbm.at[sid], sidx)
        pltpu.async_copy(h_hbm.at[gidx.at[0]], buf0, sem0)

        def body(k2, carry):
            for b in range(2):
                j = k2 * 2 + b
                pltpu.make_async_copy(h_hbm.at[gidx.at[j]], bufs[b], sems[b]).wait()

                @pl.when(j + 1 < _NCH)
                def _():
                    pltpu.async_copy(h_hbm.at[gidx.at[j + 1]], bufs[1 - b],
                                     sems[1 - b])
                pltpu.sync_copy(bufs[b], acc.at[sidx.at[j]], add=True)
            return carry
        lax.fori_loop(0, _NCH // 2, body, 0)
        plsc.subcore_barrier()
        pltpu.sync_copy(acc.at[pl.ds(r0, _RPS)], out_hbm.at[pl.ds(r0, _RPS)])

    @pl.when(cid == 0)
    def _():
        run(src2_hbm, dst2_hbm, p_hbm)

    @pl.when(cid == 1)
    def _():
        run(dst2_hbm, src2_hbm, q_hbm)

  return _spmm


def _spmm(h, src, dst, zeros_nd):
    src3 = src.reshape(16, _NCH, _CHUNK)
    dst3 = dst.reshape(16, _NCH, _CHUNK)
    return _spmm_kernel()(h, src3, dst3, zeros_nd)


def _embed_call(x3, emb16):
    def body(x_ref, emb_ref, o_ref, or_ref):
        xv = x_ref[0, 0, :]
        oh = (xv[:, None] == lax.broadcasted_iota(jnp.int32, (256, 16), 1))
        h = _dot(oh.astype(_F32), emb_ref[...])
        o_ref[...] = h
        or_ref[...] = _r(h)

    return pl.pallas_call(
        body,
        grid=(N // 256,),
        in_specs=[pl.BlockSpec((1, 1, 256), lambda i: (i, 0, 0)),
                  pl.BlockSpec((16, ND), lambda i: (0, 0))],
        out_specs=[pl.BlockSpec((256, ND), lambda i: (i, 0)),
                   pl.BlockSpec((256, ND), lambda i: (i, 0))],
        out_shape=[jax.ShapeDtypeStruct((N, ND), _F32),
                   jax.ShapeDtypeStruct((N, ND), _F32)],
    )(x3, emb16)


def _update_call(P, Q, h, hr, ind16, outd16, waT, wraT, wbT, wrbT, msgb,
                 msgrb, wihT, whhT, bih, bhh):
    R = 256

    def body(p_ref, q_ref, h_ref, hr_ref, ind_ref, outd_ref, waT_ref,
             wraT_ref, wbT_ref, wrbT_ref, msgb_ref, msgrb_ref, wihT_ref,
             whhT_ref, bih_ref, bhh_ref, o_ref, or_ref):
        hb = h_ref[...]
        hrb = hr_ref[...]
        ind = ind_ref[:, :1]
        outd = outd_ref[:, :1]
        aggr = (_dot(p_ref[...], waT_ref[...]) + _dot(q_ref[...], wraT_ref[...])
                + ind * _dot(hrb, wbT_ref[...]) + outd * _dot(hrb, wrbT_ref[...])
                + ind * msgb_ref[...] + outd * msgrb_ref[...])
        gi = _dot(_r(aggr), wihT_ref[...]) + bih_ref[...]
        gh = _dot(hrb, whhT_ref[...]) + bhh_ref[...]
        r = jax.nn.sigmoid(gi[:, :ND] + gh[:, :ND])
        z = jax.nn.sigmoid(gi[:, ND:2 * ND] + gh[:, ND:2 * ND])
        n = jnp.tanh(gi[:, 2 * ND:] + r * gh[:, 2 * ND:])
        h_new = (1.0 - z) * n + z * hb
        o_ref[...] = h_new
        or_ref[...] = _r(h_new)

    row = lambda shape: pl.BlockSpec(shape, lambda i: (i, 0))
    cst = lambda shape: pl.BlockSpec(shape, lambda i: (0, 0))
    return pl.pallas_call(
        body,
        grid=(N // R,),
        in_specs=[row((R, ND)), row((R, ND)), row((R, ND)), row((R, ND)),
                  row((R, 16)), row((R, 16)),
                  cst((ND, 2 * ND)), cst((ND, 2 * ND)),
                  cst((ND, 2 * ND)), cst((ND, 2 * ND)),
                  cst((1, 2 * ND)), cst((1, 2 * ND)),
                  cst((2 * ND, 3 * ND)), cst((ND, 3 * ND)),
                  cst((1, 3 * ND)), cst((1, 3 * ND))],
        out_specs=[row((R, ND)), row((R, ND))],
        out_shape=[jax.ShapeDtypeStruct((N, ND), _F32),
                   jax.ShapeDtypeStruct((N, ND), _F32)],
    )(P, Q, h, hr, ind16, outd16, waT, wraT, wbT, wrbT, msgb, msgrb,
      wihT, whhT, bih, bhh)


def _gated_call(h, hps8, S, batch3, fc1T8, fc1b8, fmWhT, fmWpT, fmb, gmWhT,
                gmWpT, gmb8):
    R = 240

    def body(h_ref, hp_ref, s_ref, b_ref, fc1T_ref, fc1b_ref, fmWhT_ref,
             fmWpT_ref, fmb_ref, gmWhT_ref, gmWpT_ref, gmb_ref, o_ref):
        hp = jax.nn.relu(_dot(hp_ref[...], fc1T_ref[...]) + fc1b_ref[...])
        rep = _r(_dot(s_ref[...], hp))
        hb = h_ref[...]
        hv = _dot(hb, fmWhT_ref[...]) + _dot(rep, fmWpT_ref[...]) + fmb_ref[...]
        gl = _dot(hb, gmWhT_ref[...]) + _dot(rep, gmWpT_ref[...]) + gmb_ref[...]
        gated = hv * jax.nn.sigmoid(gl[:, :1])
        bv = b_ref[0, 0, :]
        oh = (bv[:, None] == lax.broadcasted_iota(jnp.int32, (R, G), 1))
        contrib = lax.dot_general(oh.astype(_F32), gated,
                                  (((0,), (0,)), ((), ())),
                                  precision=_HI, preferred_element_type=_F32)
        i = pl.program_id(0)

        @pl.when(i == 0)
        def _():
            o_ref[...] = contrib

        @pl.when(i > 0)
        def _():
            o_ref[...] += contrib

    return pl.pallas_call(
        body,
        grid=(N // R,),
        in_specs=[pl.BlockSpec((R, ND), lambda i: (i, 0)),
                  pl.BlockSpec((8, 8), lambda i: (i, 0)),
                  pl.BlockSpec((R, 8), lambda i: (0, 0)),
                  pl.BlockSpec((1, 1, R), lambda i: (i, 0, 0)),
                  pl.BlockSpec((8, 8), lambda i: (0, 0)),
                  pl.BlockSpec((1, 8), lambda i: (0, 0)),
                  pl.BlockSpec((ND, 256), lambda i: (0, 0)),
                  pl.BlockSpec((8, 256), lambda i: (0, 0)),
                  pl.BlockSpec((1, 256), lambda i: (0, 0)),
                  pl.BlockSpec((ND, 8), lambda i: (0, 0)),
                  pl.BlockSpec((8, 8), lambda i: (0, 0)),
                  pl.BlockSpec((1, 8), lambda i: (0, 0))],
        out_specs=pl.BlockSpec((G, 256), lambda i: (0, 0)),
        out_shape=jax.ShapeDtypeStruct((G, 256), _F32),
    )(h, hps8, S, batch3, fc1T8, fc1b8, fmWhT, fmWpT, fmb, gmWhT, gmWpT, gmb8)


def _head_call(hG, w0T, b0, w1T, b1, w2T8, b28):
    def body(hg_ref, w0_ref, b0_ref, w1_ref, b1_ref, w2_ref, b2_ref, o_ref):
        o = jax.nn.relu(_dot(_r(hg_ref[...]), w0_ref[...]) + b0_ref[...])
        o = jax.nn.relu(_dot(_r(o), w1_ref[...]) + b1_ref[...])
        o_ref[...] = _dot(_r(o), w2_ref[...]) + b2_ref[...]

    return pl.pallas_call(
        body,
        out_shape=jax.ShapeDtypeStruct((G, 8), _F32),
    )(hG, w0T, b0, w1T, b1, w2T8, b28)


def kernel(x, edge_index, batch, hyperparameters, node_emb,
           msg_W0, msg_b0, msgr_W0, msgr_b0, gru_wih0, gru_whh0, gru_bih0, gru_bhh0,
           msg_W1, msg_b1, msgr_W1, msgr_b1, gru_wih1, gru_whh1, gru_bih1, gru_bhh1,
           msg_W2, msg_b2, msgr_W2, msgr_b2, gru_wih2, gru_whh2, gru_bih2, gru_bhh2,
           fc1_W, fc1_b, fm_W, fm_b, gm_W, gm_b,
           acc_W0, acc_b0, acc_W1, acc_b1, acc_W2, acc_b2):
    msg_W = [msg_W0, msg_W1, msg_W2]
    msg_b = [msg_b0, msg_b1, msg_b2]
    msgr_W = [msgr_W0, msgr_W1, msgr_W2]
    msgr_b = [msgr_b0, msgr_b1, msgr_b2]
    gru_wih = [gru_wih0, gru_wih1, gru_wih2]
    gru_whh = [gru_whh0, gru_whh1, gru_whh2]
    gru_bih = [gru_bih0, gru_bih1, gru_bih2]
    gru_bhh = [gru_bhh0, gru_bhh1, gru_bhh2]

    src = edge_index[0].astype(jnp.int32)
    dst = edge_index[1].astype(jnp.int32)

    x3 = x.astype(jnp.int32).reshape(N // 256, 1, 256)
    emb16 = jnp.zeros((16, ND), _F32).at[:11].set(node_emb)
    h, hr = _embed_call(x3, emb16)

    zeros_nd = jnp.zeros((N, ND), _F32)
    ones_nd = jnp.ones((N, ND), _F32)
    ind_full, outd_full = _spmm(ones_nd, src, dst, zeros_nd)
    ind16 = ind_full[:, :16]
    outd16 = outd_full[:, :16]

    for l in range(3):
        P, Q = _spmm(hr, src, dst, zeros_nd)
        h, hr = _update_call(
            P, Q, h, hr, ind16, outd16,
            _r(msg_W[l][:, :ND].T), _r(msgr_W[l][:, :ND].T),
            _r(msg_W[l][:, ND:].T), _r(msgr_W[l][:, ND:].T),
            msg_b[l].reshape(1, -1), msgr_b[l].reshape(1, -1),
            _r(gru_wih[l].T), _r(gru_whh[l].T),
            gru_bih[l].reshape(1, -1), gru_bhh[l].reshape(1, -1))

    hps8 = _r(jnp.zeros((G, 8), _F32).at[:, :5].set(hyperparameters.reshape(G, 5)))
    S = jnp.zeros((240, 8), _F32).at[jnp.arange(240), jnp.arange(240) // 30].set(1.0)
    fc1T8 = _r(jnp.zeros((8, 8), _F32).at[:5, :5].set(fc1_W.T))
    fc1b8 = jnp.zeros((1, 8), _F32).at[0, :5].set(fc1_b)
    fmWhT = _r(fm_W[:, :ND].T)
    fmWpT = _r(jnp.zeros((8, 256), _F32).at[:5].set(fm_W[:, ND:].T))
    fmb = fm_b.reshape(1, -1)
    gmWhT = _r(jnp.zeros((ND, 8), _F32).at[:, :1].set(gm_W[:, :ND].T))
    gmWpT = _r(jnp.zeros((8, 8), _F32).at[:5, :1].set(gm_W[:, ND:].T))
    gmb8 = jnp.zeros((1, 8), _F32).at[0, 0].set(gm_b[0])
    batch3 = batch.astype(jnp.int32).reshape(N // 240, 1, 240)
    hG = _gated_call(hr, hps8, S, batch3, fc1T8, fc1b8, fmWhT, fmWpT, fmb,
                     gmWhT, gmWpT, gmb8)

    w0T, b0 = _r(acc_W0.T), acc_b0.reshape(1, -1)
    w1T, b1 = _r(acc_W1.T), acc_b1.reshape(1, -1)
    w2T8 = _r(jnp.zeros((64, 8), _F32).at[:, :1].set(acc_W2.T))
    b28 = jnp.zeros((1, 8), _F32).at[0, 0].set(acc_b2[0])
    out8 = _head_call(hG, w0T, b0, w1T, b1, w2T8, b28)
    return out8[:, 0]
